# Initial kernel scaffold; baseline (speedup 1.0000x reference)
import jax
import jax.numpy as jnp
from jax.experimental import pallas as pl


def kernel(norm_h_x, norm_ground_level, norm_runoff, norm_elevation, edge_index, norm_edge_attr, steps_ahead, W_mix1, b_mix1, W_mix2, b_mix2, W_self, W_nbr, b_proc, W_dec, b_dec):
    raise NotImplementedError("write your pallas kernel here")


# speedup vs baseline: 4.1517x; 4.1517x over previous
import functools

import jax
import jax.numpy as jnp
from jax import lax
from jax.experimental import pallas as pl
from jax.experimental.pallas import tpu as pltpu
from jax.experimental.pallas import tpu_sc as plsc

N_NODES = 100000
N_EDGES = 1600000
HID = 16
NC = 2
NS = 16
NW = NC * NS
BLK = 128
NB = 391
E_PAD = NW * NB * BLK
N_PAD = 100352
CHUNK = N_PAD // NS
ZROWS = 784
NODE_BLK = 1024
NODE_GRID = N_PAD // NODE_BLK
EDGE_BLK = 4096
F32 = jnp.float32

_sc_mesh = plsc.VectorSubcoreMesh(
    core_axis_name="c", subcore_axis_name="s", num_cores=NC, num_subcores=NS)


def _zero_shared(zbuf, dst, sid):
    def zrow(j, _):
        zbuf[j] = jnp.zeros((16,), F32)
        return 0
    lax.fori_loop(0, BLK, zrow, 0)
    def zcp(r, _):
        pltpu.sync_copy(zbuf, dst.at[pl.ds(sid * CHUNK + r * BLK, BLK)])
        return 0
    lax.fori_loop(0, CHUNK // BLK, zcp, 0)


def _phase1_body(with_deg, A_hbm, B_hbm, eproj_hbm, src_hbm, tgt_hbm,
                 acc_out, deg_out, idxs, idxt, arows, brows, erows,
                 hrows, ones, zdeg, acc_sh, deg_sh, sem):
    cid = lax.axis_index("c")
    sid = lax.axis_index("s")
    wid = cid * NS + sid

    _zero_shared(hrows, acc_sh, sid)
    if with_deg:
        def zd(j, _):
            zdeg[pl.ds(j * 16, 16)] = jnp.zeros((16,), F32)
            return 0
        lax.fori_loop(0, ZROWS // 16, zd, 0)
        def zdc(r, _):
            pltpu.sync_copy(zdeg, deg_sh.at[pl.ds(sid * CHUNK + r * ZROWS,
                                                  ZROWS)])
            return 0
        lax.fori_loop(0, CHUNK // ZROWS, zdc, 0)
        def so(j, _):
            ones[pl.ds(j * 16, 16)] = jnp.full((16,), 1.0, F32)
            return 0
        lax.fori_loop(0, BLK // 16, so, 0)
    plsc.subcore_barrier()

    base0 = wid * NB * BLK

    def block(b, _):
        base = base0 + b * BLK
        pltpu.sync_copy(src_hbm.at[pl.ds(base, BLK)], idxs)
        pltpu.sync_copy(tgt_hbm.at[pl.ds(base, BLK)], idxt)
        ca = pltpu.async_copy(A_hbm.at[idxs], arows, sem)
        cb = pltpu.async_copy(B_hbm.at[idxt], brows, sem)
        ce = pltpu.async_copy(eproj_hbm.at[pl.ds(base, BLK)], erows, sem)
        ca.wait()
        cb.wait()
        ce.wait()

        def row(j, _):
            hrows[j] = jnp.maximum(arows[j] + brows[j] + erows[j], 0.0)
            return 0
        lax.fori_loop(0, BLK, row, 0, unroll=4)
        pltpu.sync_copy(hrows, acc_sh.at[idxs], add=True)
        if with_deg:
            pltpu.sync_copy(ones, deg_sh.at[idxs], add=True)
        return 0

    lax.fori_loop(0, NB, block, 0)
    plsc.subcore_barrier()

    lo = sid * CHUNK
    pltpu.sync_copy(acc_sh.at[pl.ds(lo, CHUNK)],
                    acc_out.at[cid, pl.ds(lo, CHUNK)])
    if with_deg:
        pltpu.sync_copy(deg_sh.at[pl.ds(lo, CHUNK)],
                        deg_out.at[cid, pl.ds(lo, CHUNK)])


def _build_phase1(with_deg):
    out_type = [jax.ShapeDtypeStruct((NC, N_PAD, HID), F32)]
    if with_deg:
        out_type.append(jax.ShapeDtypeStruct((NC, N_PAD), F32))
    scratch = [
        pltpu.VMEM((BLK,), jnp.int32),
        pltpu.VMEM((BLK,), jnp.int32),
        pltpu.VMEM((BLK, HID), F32),
        pltpu.VMEM((BLK, HID), F32),
        pltpu.VMEM((BLK, HID), F32),
        pltpu.VMEM((BLK, HID), F32),
        pltpu.VMEM((BLK,), F32),
        pltpu.VMEM((ZROWS,), F32),
        pltpu.VMEM_SHARED((N_PAD, HID), F32),
    ]
    if with_deg:
        scratch.append(pltpu.VMEM_SHARED((N_PAD,), F32))
        scratch.append(pltpu.SemaphoreType.DMA)

        def body(A, B, ep, s, t, acc_o, deg_o, *scr):
            _phase1_body(True, A, B, ep, s, t, acc_o, deg_o, *scr)
    else:
        scratch.append(pltpu.SemaphoreType.DMA)

        def body(A, B, ep, s, t, acc_o, *scr):
            _phase1_body(False, A, B, ep, s, t, acc_o, None,
                         *scr[:-1], None, scr[-1])
    return pl.kernel(body, out_type=out_type, mesh=_sc_mesh,
                     scratch_types=scratch,
                     compiler_params=pltpu.CompilerParams(
                         use_tc_tiling_on_sc=False))


_phase1_deg = _build_phase1(True)
_phase1_nodeg = _build_phase1(False)


def _phase2_body(y_hbm, src_hbm, tgt_hbm, msg_out,
                 idxs, idxt, yrows, msg_sh, sem):
    cid = lax.axis_index("c")
    sid = lax.axis_index("s")
    wid = cid * NS + sid

    _zero_shared(yrows, msg_sh, sid)
    plsc.subcore_barrier()

    base0 = wid * NB * BLK

    def block(b, _):
        base = base0 + b * BLK
        pltpu.sync_copy(src_hbm.at[pl.ds(base, BLK)], idxs)
        pltpu.sync_copy(tgt_hbm.at[pl.ds(base, BLK)], idxt)
        pltpu.async_copy(y_hbm.at[idxs], yrows, sem).wait()
        pltpu.sync_copy(yrows, msg_sh.at[idxt], add=True)
        return 0

    lax.fori_loop(0, NB, block, 0)
    plsc.subcore_barrier()

    lo = sid * CHUNK
    pltpu.sync_copy(msg_sh.at[pl.ds(lo, CHUNK)],
                    msg_out.at[cid, pl.ds(lo, CHUNK)])


_phase2 = pl.kernel(
    _phase2_body,
    out_type=jax.ShapeDtypeStruct((NC, N_PAD, HID), F32),
    mesh=_sc_mesh,
    scratch_types=[
        pltpu.VMEM((BLK,), jnp.int32),
        pltpu.VMEM((BLK,), jnp.int32),
        pltpu.VMEM((BLK, HID), F32),
        pltpu.VMEM_SHARED((N_PAD, HID), F32),
        pltpu.SemaphoreType.DMA,
    ],
    compiler_params=pltpu.CompilerParams(use_tc_tiling_on_sc=False))


def _eproj_kbody(ea_ref, w_ref, out_ref):
    out_ref[...] = jnp.dot(ea_ref[...], w_ref[...],
                           preferred_element_type=F32,
                           precision=jax.lax.Precision.HIGHEST)


def _eproj_call(ea_pad, W1e):
    return pl.pallas_call(
        _eproj_kbody,
        grid=(E_PAD // EDGE_BLK,),
        in_specs=[
            pl.BlockSpec((EDGE_BLK, 4), lambda i: (i, 0)),
            pl.BlockSpec((4, HID), lambda i: (0, 0)),
        ],
        out_specs=pl.BlockSpec((EDGE_BLK, HID), lambda i: (i, 0)),
        out_shape=jax.ShapeDtypeStruct((E_PAD, HID), F32),
    )(ea_pad, W1e)


def _prep_kbody(h0_ref, run_ref, elev_ref, ws_ref, wt_ref, b1_ref,
                a_ref, b_ref):
    nf = jnp.concatenate([h0_ref[...], run_ref[...], elev_ref[...]], axis=1)
    a_ref[...] = jnp.dot(nf, ws_ref[...], preferred_element_type=F32,
                           precision=jax.lax.Precision.HIGHEST) + b1_ref[...]
    b_ref[...] = jnp.dot(nf, wt_ref[...], preferred_element_type=F32,
                           precision=jax.lax.Precision.HIGHEST)


def _prep_call(h0, run6, elev, W1s, W1t, b1):
    return pl.pallas_call(
        _prep_kbody,
        grid=(NODE_GRID,),
        in_specs=[
            pl.BlockSpec((NODE_BLK, 4), lambda i: (i, 0)),
            pl.BlockSpec((NODE_BLK, 6), lambda i: (i, 0)),
            pl.BlockSpec((NODE_BLK, 1), lambda i: (i, 0)),
            pl.BlockSpec((11, HID), lambda i: (0, 0)),
            pl.BlockSpec((11, HID), lambda i: (0, 0)),
            pl.BlockSpec((1, HID), lambda i: (0, 0)),
        ],
        out_specs=[
            pl.BlockSpec((NODE_BLK, HID), lambda i: (i, 0)),
            pl.BlockSpec((NODE_BLK, HID), lambda i: (i, 0)),
        ],
        out_shape=[
            jax.ShapeDtypeStruct((N_PAD, HID), F32),
            jax.ShapeDtypeStruct((N_PAD, HID), F32),
        ],
    )(h0, run6, elev, W1s, W1t, b1)


def _mid_kbody(acc_ref, deg_ref, w2_ref, b2_ref, wn_ref, ws_ref,
               y_ref, zs_ref):
    a = acc_ref[0] + acc_ref[1]
    d = deg_ref[0] + deg_ref[1]
    cx = (jnp.dot(a, w2_ref[...], preferred_element_type=F32,
                           precision=jax.lax.Precision.HIGHEST)
          + d[:, None] * b2_ref[...])
    y_ref[...] = jnp.dot(cx, wn_ref[...], preferred_element_type=F32,
                           precision=jax.lax.Precision.HIGHEST)
    zs_ref[...] = jnp.dot(cx, ws_ref[...], preferred_element_type=F32,
                           precision=jax.lax.Precision.HIGHEST)


def _mid_call(acc, deg, W2, b2, Wn, Ws):
    return pl.pallas_call(
        _mid_kbody,
        grid=(NODE_GRID,),
        in_specs=[
            pl.BlockSpec((NC, NODE_BLK, HID), lambda i: (0, i, 0)),
            pl.BlockSpec((NC, NODE_BLK), lambda i: (0, i)),
            pl.BlockSpec((HID, HID), lambda i: (0, 0)),
            pl.BlockSpec((1, HID), lambda i: (0, 0)),
            pl.BlockSpec((HID, HID), lambda i: (0, 0)),
            pl.BlockSpec((HID, HID), lambda i: (0, 0)),
        ],
        out_specs=[
            pl.BlockSpec((NODE_BLK, HID), lambda i: (i, 0)),
            pl.BlockSpec((NODE_BLK, HID), lambda i: (i, 0)),
        ],
        out_shape=[
            jax.ShapeDtypeStruct((N_PAD, HID), F32),
            jax.ShapeDtypeStruct((N_PAD, HID), F32),
        ],
    )(acc, deg, W2, b2, Wn, Ws)


def _post_kbody(compute_next, zs_ref, msg_ref, bp_ref, wd_ref, bd_ref,
                h0_ref, elev_ref, gl_ref, runn_ref, ws_ref, wt_ref, b1_ref,
                mask_ref, pred_ref, h0n_ref, a_ref, b_ref):
    processed = jnp.maximum(
        zs_ref[...] + msg_ref[0] + msg_ref[1] + bp_ref[...], 0.0)
    dec = jnp.dot(processed, wd_ref[...], preferred_element_type=F32,
                           precision=jax.lax.Precision.HIGHEST) + bd_ref[...]
    elev = elev_ref[...]
    prev_y = h0_ref[:, 3:4] - elev
    ph = 0.5 * dec + 0.5 * prev_y + elev
    phc = jnp.minimum(ph, gl_ref[...])
    phlc = jnp.maximum(phc, elev)
    pred_ref[...] = phlc * mask_ref[...]
    if compute_next:
        h0n = jnp.concatenate([h0_ref[:, 2:4], phc], axis=1)
        h0n_ref[...] = h0n
        nf = jnp.concatenate([h0n, runn_ref[...], elev], axis=1)
        a_ref[...] = jnp.dot(nf, ws_ref[...], preferred_element_type=F32,
                           precision=jax.lax.Precision.HIGHEST) + b1_ref[...]
        b_ref[...] = jnp.dot(nf, wt_ref[...], preferred_element_type=F32,
                           precision=jax.lax.Precision.HIGHEST)


def _post_call(compute_next, zs, msg, bp, Wd, bd, h0, elev, gl, runn,
               W1s, W1t, b1, mask):
    out_specs = [pl.BlockSpec((NODE_BLK, 2), lambda i: (i, 0))]
    out_shape = [jax.ShapeDtypeStruct((N_PAD, 2), F32)]
    if compute_next:
        out_specs += [
            pl.BlockSpec((NODE_BLK, 4), lambda i: (i, 0)),
            pl.BlockSpec((NODE_BLK, HID), lambda i: (i, 0)),
            pl.BlockSpec((NODE_BLK, HID), lambda i: (i, 0)),
        ]
        out_shape += [
            jax.ShapeDtypeStruct((N_PAD, 4), F32),
            jax.ShapeDtypeStruct((N_PAD, HID), F32),
            jax.ShapeDtypeStruct((N_PAD, HID), F32),
        ]
        body = functools.partial(_post_kbody, True)
    else:
        def body(*refs):
            _post_kbody(False, *refs[:14], None, None, None)
    return pl.pallas_call(
        body,
        grid=(NODE_GRID,),
        in_specs=[
            pl.BlockSpec((NODE_BLK, HID), lambda i: (i, 0)),
            pl.BlockSpec((NC, NODE_BLK, HID), lambda i: (0, i, 0)),
            pl.BlockSpec((1, HID), lambda i: (0, 0)),
            pl.BlockSpec((HID, 2), lambda i: (0, 0)),
            pl.BlockSpec((1, 2), lambda i: (0, 0)),
            pl.BlockSpec((NODE_BLK, 4), lambda i: (i, 0)),
            pl.BlockSpec((NODE_BLK, 1), lambda i: (i, 0)),
            pl.BlockSpec((NODE_BLK, 1), lambda i: (i, 0)),
            pl.BlockSpec((NODE_BLK, 6), lambda i: (i, 0)),
            pl.BlockSpec((11, HID), lambda i: (0, 0)),
            pl.BlockSpec((11, HID), lambda i: (0, 0)),
            pl.BlockSpec((1, HID), lambda i: (0, 0)),
            pl.BlockSpec((1, 2), lambda i: (0, 0)),
        ],
        out_specs=out_specs,
        out_shape=out_shape,
    )(zs, msg, bp, Wd, bd, h0, elev, gl, runn, W1s, W1t, b1, mask)


def kernel(norm_h_x, norm_ground_level, norm_runoff, norm_elevation,
           edge_index, norm_edge_attr, steps_ahead,
           W_mix1, b_mix1, W_mix2, b_mix2,
           W_self, W_nbr, b_proc, W_dec, b_dec):
    n = norm_h_x.shape[0]
    pad_n = ((0, N_PAD - n), (0, 0))
    h0 = jnp.pad(norm_h_x, pad_n)
    gl = jnp.pad(norm_ground_level, pad_n)
    elev = jnp.pad(norm_elevation, pad_n)
    runoff = jnp.pad(norm_runoff, pad_n)

    src = edge_index[0].astype(jnp.int32)
    tgt = edge_index[1].astype(jnp.int32)
    epad = E_PAD - src.shape[0]
    src = jnp.concatenate([src, jnp.full((epad,), n, jnp.int32)])
    tgt = jnp.concatenate([tgt, jnp.full((epad,), n, jnp.int32)])
    ea_pad = jnp.pad(norm_edge_attr, ((0, epad), (0, 0)))

    W1e = W_mix1[0:4]
    W1s = W_mix1[4:15]
    W1t = W_mix1[15:26]
    b1 = b_mix1.reshape(1, HID)
    b2 = b_mix2.reshape(1, HID)
    bp = b_proc.reshape(1, HID)
    bd = b_dec.reshape(1, 2)

    eproj = _eproj_call(ea_pad, W1e)
    A, B = _prep_call(h0[:, :4], runoff[:, 0:6], elev, W1s, W1t, b1)

    steps_static = norm_runoff.shape[1] - 4
    preds = []
    deg = None
    for step in range(0, steps_static, 2):
        if deg is None:
            acc, deg = _phase1_deg(A, B, eproj, src, tgt)
        else:
            (acc,) = _phase1_nodeg(A, B, eproj, src, tgt)
        y, zs = _mid_call(acc, deg, W_mix2, b2, W_nbr, W_self)
        msg = _phase2(y, src, tgt)
        mask = ((step + jnp.arange(2)) < steps_ahead).astype(F32).reshape(1, 2)
        last = step + 2 >= steps_static
        if last:
            (pred,) = _post_call(False, zs, msg, bp, W_dec, bd, h0, elev, gl,
                                 runoff[:, 0:6], W1s, W1t, b1, mask)
        else:
            pred, h0, A, B = _post_call(
                True, zs, msg, bp, W_dec, bd, h0, elev, gl,
                runoff[:, step + 2:step + 8], W1s, W1t, b1, mask)
        preds.append(pred[:n])
    return jnp.concatenate(preds, axis=1)
